# baseline (speedup 1.0000x reference)
import functools

import jax
import jax.numpy as jnp
from jax import lax
from jax.experimental import pallas as pl
from jax.experimental.pallas import tpu as pltpu
from jax.experimental.pallas import tpu_sc as plsc

NN = 10000
NP = 10240
D = 128
H = 128
E = 320000
NW = 32
NS = 16
KCH0 = 160
KCH1 = 0
PART = 40
MAXC = 160
NCH = NS * (KCH0 + KCH1)
NCHP = 2720
EPAD = NCH * 128
RPT = NP // NS
RB = 640
EB = 2048


@functools.lru_cache(maxsize=None)
def _sc_kernels():
    mesh = plsc.VectorSubcoreMesh(core_axis_name="c", subcore_axis_name="s")

    def _range(cid, sid):
        crow = jnp.where(cid == 0, sid * KCH0, NS * KCH0 + sid * KCH1)
        nch = jnp.where(cid == 0, KCH0, KCH1)
        return crow, nch

    @functools.partial(
        pl.kernel,
        out_type=jax.ShapeDtypeStruct((2, NP, H), jnp.float32),
        mesh=mesh,
        scratch_types=[
            pltpu.VMEM((MAXC, 128), jnp.int32),
            pltpu.VMEM((128, H), jnp.float32),
            pltpu.VMEM_SHARED((NP, H), jnp.float32),
        ],
    )
    def deg_kernel(dst_hbm, zeros_hbm, ones_hbm, degp_hbm, idx_v, ones_v, shared_deg):
        cid = lax.axis_index("c")
        sid = lax.axis_index("s")
        crow, nch = _range(cid, sid)
        pltpu.sync_copy(ones_hbm, ones_v)
        pltpu.sync_copy(zeros_hbm, shared_deg.at[pl.ds(sid * RPT, RPT)])
        pltpu.sync_copy(dst_hbm.at[pl.ds(crow, MAXC)], idx_v)
        plsc.subcore_barrier()

        def body(j, carry):
            pltpu.sync_copy(ones_v, shared_deg.at[idx_v.at[j]], add=True)
            return carry

        lax.fori_loop(0, nch, body, 0)
        plsc.subcore_barrier()
        pltpu.sync_copy(shared_deg.at[pl.ds(sid * RPT, RPT)],
                        degp_hbm.at[cid, pl.ds(sid * RPT, RPT)])

    @functools.partial(
        pl.kernel,
        out_type=jax.ShapeDtypeStruct((2, NP, H), jnp.float32),
        mesh=mesh,
        scratch_types=[
            pltpu.VMEM((PART, 128), jnp.int32),
            pltpu.VMEM((PART, 128), jnp.int32),
            pltpu.VMEM((2, 128, H), jnp.float32),
            pltpu.VMEM_SHARED((NP, H), jnp.float32),
            pltpu.SemaphoreType.DMA,
            pltpu.SemaphoreType.DMA,
        ],
    )
    def agg_kernel(y_hbm, src_hbm, dst_hbm, zeros_hbm, aggp_hbm,
                   src_v, dst_v, rows_v, shared_agg, sem0, sem1):
        cid = lax.axis_index("c")
        sid = lax.axis_index("s")
        crow, nch = _range(cid, sid)
        pp = PART // 2
        pltpu.sync_copy(zeros_hbm, shared_agg.at[pl.ds(sid * RPT, RPT)])
        plsc.subcore_barrier()

        def part_body(part, pcarry):

            @pl.when(part * PART < nch)
            def _():
                pltpu.sync_copy(src_hbm.at[pl.ds(crow + part * PART, PART)], src_v)
                pltpu.sync_copy(dst_hbm.at[pl.ds(crow + part * PART, PART)], dst_v)
                pltpu.async_copy(y_hbm.at[src_v.at[0]], rows_v.at[0], sem0)

                def body(jj, carry):
                    j0 = 2 * jj
                    j1 = j0 + 1
                    pltpu.async_copy(y_hbm.at[src_v.at[j1]], rows_v.at[1], sem1)
                    pltpu.make_async_copy(y_hbm.at[src_v.at[j0]], rows_v.at[0], sem0).wait()
                    pltpu.sync_copy(rows_v.at[0], shared_agg.at[dst_v.at[j0]], add=True)

                    @pl.when(jj + 1 < pp)
                    def _():
                        pltpu.async_copy(y_hbm.at[src_v.at[j0 + 2]], rows_v.at[0], sem0)

                    pltpu.make_async_copy(y_hbm.at[src_v.at[j1]], rows_v.at[1], sem1).wait()
                    pltpu.sync_copy(rows_v.at[1], shared_agg.at[dst_v.at[j1]], add=True)
                    return carry

                lax.fori_loop(0, pp, body, 0)

            return pcarry

        lax.fori_loop(0, MAXC // PART, part_body, 0)

        plsc.subcore_barrier()
        pltpu.sync_copy(shared_agg.at[pl.ds(sid * RPT, RPT)],
                        aggp_hbm.at[cid, pl.ds(sid * RPT, RPT)])

    @functools.partial(
        pl.kernel,
        out_type=(
            jax.ShapeDtypeStruct((EPAD, H), jnp.float32),
            jax.ShapeDtypeStruct((EPAD, H), jnp.float32),
        ),
        mesh=mesh,
        scratch_types=[
            pltpu.VMEM((MAXC, 128), jnp.int32),
            pltpu.VMEM((MAXC, 128), jnp.int32),
            pltpu.VMEM((2, 128, H), jnp.float32),
            pltpu.VMEM((2, 128, H), jnp.float32),
            pltpu.SemaphoreType.DMA,
            pltpu.SemaphoreType.DMA,
            pltpu.SemaphoreType.DMA,
            pltpu.SemaphoreType.DMA,
            pltpu.SemaphoreType.DMA,
            pltpu.SemaphoreType.DMA,
            pltpu.SemaphoreType.DMA,
            pltpu.SemaphoreType.DMA,
        ],
    )
    def edge_kernel(a_hbm, b_hbm, src_hbm, dst_hbm, ga_hbm, gb_hbm,
                    src_v, dst_v, ra_v, rb_v, sa0, sb0, sa1, sb1, swa0, swa1, swb0, swb1):
        cid = lax.axis_index("c")
        sid = lax.axis_index("s")
        crow, nch = _range(cid, sid)
        base = crow * 128
        pltpu.sync_copy(src_hbm.at[pl.ds(crow, MAXC)], src_v)
        pltpu.sync_copy(dst_hbm.at[pl.ds(crow, MAXC)], dst_v)

        @pl.when(nch > 0)
        def _():
            pltpu.async_copy(a_hbm.at[src_v.at[0]], ra_v.at[0], sa0)
            pltpu.async_copy(b_hbm.at[dst_v.at[0]], rb_v.at[0], sb0)
            pltpu.async_copy(a_hbm.at[src_v.at[1]], ra_v.at[1], sa1)
            pltpu.async_copy(b_hbm.at[dst_v.at[1]], rb_v.at[1], sb1)

        def body(jj, carry):
            j0 = 2 * jj
            j1 = j0 + 1
            pltpu.make_async_copy(a_hbm.at[src_v.at[j0]], ra_v.at[0], sa0).wait()
            pltpu.async_copy(ra_v.at[0], ga_hbm.at[pl.ds(base + j0 * 128, 128)], swa0)
            pltpu.make_async_copy(b_hbm.at[dst_v.at[j0]], rb_v.at[0], sb0).wait()
            pltpu.async_copy(rb_v.at[0], gb_hbm.at[pl.ds(base + j0 * 128, 128)], swb0)
            pltpu.make_async_copy(a_hbm.at[src_v.at[j1]], ra_v.at[1], sa1).wait()
            pltpu.async_copy(ra_v.at[1], ga_hbm.at[pl.ds(base + j1 * 128, 128)], swa1)
            pltpu.make_async_copy(b_hbm.at[dst_v.at[j1]], rb_v.at[1], sb1).wait()
            pltpu.async_copy(rb_v.at[1], gb_hbm.at[pl.ds(base + j1 * 128, 128)], swb1)

            @pl.when(jj + 1 < nch // 2)
            def _():
                pltpu.make_async_copy(ra_v.at[0], ga_hbm.at[pl.ds(base, 128)], swa0).wait()
                pltpu.async_copy(a_hbm.at[src_v.at[j0 + 2]], ra_v.at[0], sa0)
                pltpu.make_async_copy(rb_v.at[0], gb_hbm.at[pl.ds(base, 128)], swb0).wait()
                pltpu.async_copy(b_hbm.at[dst_v.at[j0 + 2]], rb_v.at[0], sb0)
                pltpu.make_async_copy(ra_v.at[1], ga_hbm.at[pl.ds(base, 128)], swa1).wait()
                pltpu.async_copy(a_hbm.at[src_v.at[j1 + 2]], ra_v.at[1], sa1)
                pltpu.make_async_copy(rb_v.at[1], gb_hbm.at[pl.ds(base, 128)], swb1).wait()
                pltpu.async_copy(b_hbm.at[dst_v.at[j1 + 2]], rb_v.at[1], sb1)

            return carry

        lax.fori_loop(0, nch // 2, body, 0)

        @pl.when(nch > 0)
        def _():
            pltpu.make_async_copy(ra_v.at[0], ga_hbm.at[pl.ds(base, 128)], swa0).wait()
            pltpu.make_async_copy(ra_v.at[1], ga_hbm.at[pl.ds(base, 128)], swa1).wait()
            pltpu.make_async_copy(rb_v.at[0], gb_hbm.at[pl.ds(base, 128)], swb0).wait()
            pltpu.make_async_copy(rb_v.at[1], gb_hbm.at[pl.ds(base, 128)], swb1).wait()

    return deg_kernel, agg_kernel, edge_kernel


def _t1_body(degp_ref, x_ref, w_ref, dinv_ref, y_ref):
    i = pl.program_id(0)
    dp = degp_ref[...]
    deg = dp[0, :, :16] + dp[1, :, :16] + 1.0
    rows = lax.broadcasted_iota(jnp.int32, (RB, 16), 0) + i * RB
    dinv = jnp.where(rows < NN, lax.rsqrt(deg), 0.0)
    dinv_ref[...] = dinv
    y_ref[...] = (x_ref[...] @ w_ref[...]) * dinv[:, :1]


def _t2_body(aggp_ref, y_ref, dinv_ref, b_ref, t_ref, stats_ref):
    i = pl.program_id(0)
    ap = aggp_ref[...]
    tt = (ap[0] + ap[1] + y_ref[...]) * dinv_ref[...][:, :1] + b_ref[...][None, :]
    rows = lax.broadcasted_iota(jnp.int32, (RB, H), 0) + i * RB
    ttm = jnp.where(rows < NN, tt, 0.0)
    s0 = jnp.sum(ttm, axis=0)
    s1 = jnp.sum(ttm * ttm, axis=0)
    st = jnp.concatenate([s0[None, :], s1[None, :]], axis=0)
    t_ref[...] = tt

    @pl.when(i == 0)
    def _():
        stats_ref[...] = st

    @pl.when(i > 0)
    def _():
        stats_ref[...] = stats_ref[...] + st


def _bn_relu(t_ref, stats_ref, g_ref, be_ref):
    st = stats_ref[...]
    m = st[0] / NN
    v = st[1] / NN - m * m
    s = lax.rsqrt(v + 1e-5) * g_ref[...]
    c = be_ref[...] - m * s
    return jnp.maximum(t_ref[...] * s[None, :] + c[None, :], 0.0)


def _t3_body(t_ref, stats_ref, g_ref, be_ref, w_ref, dinv_ref, y_ref):
    xb = _bn_relu(t_ref, stats_ref, g_ref, be_ref)
    y_ref[...] = (xb @ w_ref[...]) * dinv_ref[...][:, :1]


def _t7_body(t_ref, stats_ref, g_ref, be_ref, wc0_ref, bc0_ref, a_ref, b_ref):
    xb = _bn_relu(t_ref, stats_ref, g_ref, be_ref)
    w = wc0_ref[...]
    a_ref[...] = xb @ w[:D, :] + bc0_ref[...][None, :]
    b_ref[...] = xb @ w[D:, :]


def _t8_body(ga_ref, gb_ref, wc1_ref, bc1_ref, wc2_ref, bc2_ref, o_ref):
    h0 = jnp.maximum(ga_ref[...] + gb_ref[...], 0.0)
    h1 = jnp.maximum(h0 @ wc1_ref[...] + bc1_ref[...][None, :], 0.0)
    o_ref[...] = h1 @ wc2_ref[...] + bc2_ref[...][None, :]


def _full(shape):
    nd = len(shape)
    return pl.BlockSpec(shape, lambda i: (0,) * nd)


def kernel(x, edge_index, W0, b0, g0, be0, W1, b1, g1, be1, W2, b2, g2, be2,
           Wc0, bc0, Wc1, bc1, Wc2, bc2):
    f32 = jnp.float32
    x_p = jnp.pad(x, ((0, NP - NN), (0, 0)))
    src = jnp.pad(edge_index[0], (0, NCHP * 128 - E), constant_values=NN).reshape(NCHP, 128)
    dst = jnp.pad(edge_index[1], (0, NCHP * 128 - E), constant_values=NN).reshape(NCHP, 128)
    zeros_h = jnp.zeros((RPT, H), f32)
    ones_h = jnp.ones((128, H), f32)

    ngrid = NP // RB

    _deg_kernel, _agg_kernel, _edge_kernel = _sc_kernels()

    degp = _deg_kernel(dst, zeros_h, ones_h)

    dinv, y = pl.pallas_call(
        _t1_body,
        grid=(ngrid,),
        in_specs=[
            pl.BlockSpec((2, RB, H), lambda i: (0, i, 0)),
            pl.BlockSpec((RB, D), lambda i: (i, 0)),
            _full((D, H)),
        ],
        out_specs=[
            pl.BlockSpec((RB, 16), lambda i: (i, 0)),
            pl.BlockSpec((RB, H), lambda i: (i, 0)),
        ],
        out_shape=[
            jax.ShapeDtypeStruct((NP, 16), f32),
            jax.ShapeDtypeStruct((NP, H), f32),
        ],
    )(degp, x_p, W0)

    t2_call = pl.pallas_call(
        _t2_body,
        grid=(ngrid,),
        in_specs=[
            pl.BlockSpec((2, RB, H), lambda i: (0, i, 0)),
            pl.BlockSpec((RB, H), lambda i: (i, 0)),
            pl.BlockSpec((RB, 16), lambda i: (i, 0)),
            _full((H,)),
        ],
        out_specs=[
            pl.BlockSpec((RB, H), lambda i: (i, 0)),
            _full((2, H)),
        ],
        out_shape=[
            jax.ShapeDtypeStruct((NP, H), f32),
            jax.ShapeDtypeStruct((2, H), f32),
        ],
    )

    t3_call = pl.pallas_call(
        _t3_body,
        grid=(ngrid,),
        in_specs=[
            pl.BlockSpec((RB, H), lambda i: (i, 0)),
            _full((2, H)),
            _full((H,)),
            _full((H,)),
            _full((H, H)),
            pl.BlockSpec((RB, 16), lambda i: (i, 0)),
        ],
        out_specs=pl.BlockSpec((RB, H), lambda i: (i, 0)),
        out_shape=jax.ShapeDtypeStruct((NP, H), f32),
    )

    for lidx, (bl, gl, bel, Wn) in enumerate(
            ((b0, g0, be0, W1), (b1, g1, be1, W2))):
        aggp = _agg_kernel(y, src, dst, zeros_h)
        t, stats = t2_call(aggp, y, dinv, bl)
        y = t3_call(t, stats, gl, bel, Wn, dinv)

    aggp = _agg_kernel(y, src, dst, zeros_h)
    t, stats = t2_call(aggp, y, dinv, b2)

    A, B = pl.pallas_call(
        _t7_body,
        grid=(ngrid,),
        in_specs=[
            pl.BlockSpec((RB, H), lambda i: (i, 0)),
            _full((2, H)),
            _full((H,)),
            _full((H,)),
            _full((2 * H, H)),
            _full((H,)),
        ],
        out_specs=[
            pl.BlockSpec((RB, H), lambda i: (i, 0)),
            pl.BlockSpec((RB, H), lambda i: (i, 0)),
        ],
        out_shape=[
            jax.ShapeDtypeStruct((NP, H), f32),
            jax.ShapeDtypeStruct((NP, H), f32),
        ],
    )(t, stats, g2, be2, Wc0, bc0)

    ga, gb = _edge_kernel(A, B, src, dst)

    out = pl.pallas_call(
        _t8_body,
        grid=(EPAD // EB,),
        in_specs=[
            pl.BlockSpec((EB, H), lambda i: (i, 0)),
            pl.BlockSpec((EB, H), lambda i: (i, 0)),
            _full((H, H // 2)),
            _full((H // 2,)),
            _full((H // 2, 2)),
            _full((2,)),
        ],
        out_specs=pl.BlockSpec((EB, 2), lambda i: (i, 0)),
        out_shape=jax.ShapeDtypeStruct((EPAD, 2), f32),
    )(ga, gb, Wc1, bc1, Wc2, bc2)

    return out[:E]


# speedup vs baseline: 1.3677x; 1.3677x over previous
import functools

import jax
import jax.numpy as jnp
from jax import lax
from jax.experimental import pallas as pl
from jax.experimental.pallas import tpu as pltpu
from jax.experimental.pallas import tpu_sc as plsc

NN = 10000
NP = 10240
D = 128
H = 128
E = 320000
NW = 32
NS = 16
KCH0 = 120
KCH1 = 40
PART = 40
MAXC = 120
NCH = NS * (KCH0 + KCH1)
NCHP = 2688
EPAD = NCH * 128
RPT = NP // NS
RB = 640
EB = 2048


@functools.lru_cache(maxsize=None)
def _sc_kernels():
    mesh = plsc.VectorSubcoreMesh(core_axis_name="c", subcore_axis_name="s")

    def _range(cid, sid):
        crow = jnp.where(cid == 0, sid * KCH0, NS * KCH0 + sid * KCH1)
        nch = jnp.where(cid == 0, KCH0, KCH1)
        return crow, nch

    @functools.partial(
        pl.kernel,
        out_type=jax.ShapeDtypeStruct((2, NP, H), jnp.float32),
        mesh=mesh,
        scratch_types=[
            pltpu.VMEM((MAXC, 128), jnp.int32),
            pltpu.VMEM((128, H), jnp.float32),
            pltpu.VMEM_SHARED((NP, H), jnp.float32),
        ],
    )
    def deg_kernel(dst_hbm, zeros_hbm, ones_hbm, degp_hbm, idx_v, ones_v, shared_deg):
        cid = lax.axis_index("c")
        sid = lax.axis_index("s")
        crow, nch = _range(cid, sid)
        pltpu.sync_copy(ones_hbm, ones_v)
        pltpu.sync_copy(zeros_hbm, shared_deg.at[pl.ds(sid * RPT, RPT)])
        pltpu.sync_copy(dst_hbm.at[pl.ds(crow, MAXC)], idx_v)
        plsc.subcore_barrier()

        def body(j, carry):
            pltpu.sync_copy(ones_v, shared_deg.at[idx_v.at[j]], add=True)
            return carry

        lax.fori_loop(0, nch, body, 0)
        plsc.subcore_barrier()
        pltpu.sync_copy(shared_deg.at[pl.ds(sid * RPT, RPT)],
                        degp_hbm.at[cid, pl.ds(sid * RPT, RPT)])

    @functools.partial(
        pl.kernel,
        out_type=jax.ShapeDtypeStruct((2, NP, H), jnp.float32),
        mesh=mesh,
        scratch_types=[
            pltpu.VMEM((PART, 128), jnp.int32),
            pltpu.VMEM((PART, 128), jnp.int32),
            pltpu.VMEM((2, 128, H), jnp.float32),
            pltpu.VMEM_SHARED((NP, H), jnp.float32),
            pltpu.SemaphoreType.DMA,
            pltpu.SemaphoreType.DMA,
        ],
    )
    def agg_kernel(y_hbm, src_hbm, dst_hbm, zeros_hbm, aggp_hbm,
                   src_v, dst_v, rows_v, shared_agg, sem0, sem1):
        cid = lax.axis_index("c")
        sid = lax.axis_index("s")
        crow, nch = _range(cid, sid)
        pp = PART // 2
        pltpu.sync_copy(zeros_hbm, shared_agg.at[pl.ds(sid * RPT, RPT)])
        plsc.subcore_barrier()

        def part_body(part, pcarry):

            @pl.when(part * PART < nch)
            def _():
                pltpu.sync_copy(src_hbm.at[pl.ds(crow + part * PART, PART)], src_v)
                pltpu.sync_copy(dst_hbm.at[pl.ds(crow + part * PART, PART)], dst_v)
                pltpu.async_copy(y_hbm.at[src_v.at[0]], rows_v.at[0], sem0)

                def body(jj, carry):
                    j0 = 2 * jj
                    j1 = j0 + 1
                    pltpu.async_copy(y_hbm.at[src_v.at[j1]], rows_v.at[1], sem1)
                    pltpu.make_async_copy(y_hbm.at[src_v.at[j0]], rows_v.at[0], sem0).wait()
                    pltpu.sync_copy(rows_v.at[0], shared_agg.at[dst_v.at[j0]], add=True)

                    @pl.when(jj + 1 < pp)
                    def _():
                        pltpu.async_copy(y_hbm.at[src_v.at[j0 + 2]], rows_v.at[0], sem0)

                    pltpu.make_async_copy(y_hbm.at[src_v.at[j1]], rows_v.at[1], sem1).wait()
                    pltpu.sync_copy(rows_v.at[1], shared_agg.at[dst_v.at[j1]], add=True)
                    return carry

                lax.fori_loop(0, pp, body, 0)

            return pcarry

        lax.fori_loop(0, MAXC // PART, part_body, 0)

        plsc.subcore_barrier()
        pltpu.sync_copy(shared_agg.at[pl.ds(sid * RPT, RPT)],
                        aggp_hbm.at[cid, pl.ds(sid * RPT, RPT)])

    @functools.partial(
        pl.kernel,
        out_type=(
            jax.ShapeDtypeStruct((EPAD, H), jnp.float32),
            jax.ShapeDtypeStruct((EPAD, H), jnp.float32),
        ),
        mesh=mesh,
        scratch_types=[
            pltpu.VMEM((MAXC, 128), jnp.int32),
            pltpu.VMEM((MAXC, 128), jnp.int32),
            pltpu.VMEM((2, 128, H), jnp.float32),
            pltpu.VMEM((2, 128, H), jnp.float32),
            pltpu.SemaphoreType.DMA,
            pltpu.SemaphoreType.DMA,
            pltpu.SemaphoreType.DMA,
            pltpu.SemaphoreType.DMA,
            pltpu.SemaphoreType.DMA,
            pltpu.SemaphoreType.DMA,
            pltpu.SemaphoreType.DMA,
            pltpu.SemaphoreType.DMA,
        ],
    )
    def edge_kernel(a_hbm, b_hbm, src_hbm, dst_hbm, ga_hbm, gb_hbm,
                    src_v, dst_v, ra_v, rb_v, sa0, sb0, sa1, sb1, swa0, swa1, swb0, swb1):
        cid = lax.axis_index("c")
        sid = lax.axis_index("s")
        crow, nch = _range(cid, sid)
        base = crow * 128
        pltpu.sync_copy(src_hbm.at[pl.ds(crow, MAXC)], src_v)
        pltpu.sync_copy(dst_hbm.at[pl.ds(crow, MAXC)], dst_v)

        @pl.when(nch > 0)
        def _():
            pltpu.async_copy(a_hbm.at[src_v.at[0]], ra_v.at[0], sa0)
            pltpu.async_copy(b_hbm.at[dst_v.at[0]], rb_v.at[0], sb0)
            pltpu.async_copy(a_hbm.at[src_v.at[1]], ra_v.at[1], sa1)
            pltpu.async_copy(b_hbm.at[dst_v.at[1]], rb_v.at[1], sb1)

        def body(jj, carry):
            j0 = 2 * jj
            j1 = j0 + 1
            pltpu.make_async_copy(a_hbm.at[src_v.at[j0]], ra_v.at[0], sa0).wait()
            pltpu.async_copy(ra_v.at[0], ga_hbm.at[pl.ds(base + j0 * 128, 128)], swa0)
            pltpu.make_async_copy(b_hbm.at[dst_v.at[j0]], rb_v.at[0], sb0).wait()
            pltpu.async_copy(rb_v.at[0], gb_hbm.at[pl.ds(base + j0 * 128, 128)], swb0)
            pltpu.make_async_copy(a_hbm.at[src_v.at[j1]], ra_v.at[1], sa1).wait()
            pltpu.async_copy(ra_v.at[1], ga_hbm.at[pl.ds(base + j1 * 128, 128)], swa1)
            pltpu.make_async_copy(b_hbm.at[dst_v.at[j1]], rb_v.at[1], sb1).wait()
            pltpu.async_copy(rb_v.at[1], gb_hbm.at[pl.ds(base + j1 * 128, 128)], swb1)

            @pl.when(jj + 1 < nch // 2)
            def _():
                pltpu.make_async_copy(ra_v.at[0], ga_hbm.at[pl.ds(base, 128)], swa0).wait()
                pltpu.async_copy(a_hbm.at[src_v.at[j0 + 2]], ra_v.at[0], sa0)
                pltpu.make_async_copy(rb_v.at[0], gb_hbm.at[pl.ds(base, 128)], swb0).wait()
                pltpu.async_copy(b_hbm.at[dst_v.at[j0 + 2]], rb_v.at[0], sb0)
                pltpu.make_async_copy(ra_v.at[1], ga_hbm.at[pl.ds(base, 128)], swa1).wait()
                pltpu.async_copy(a_hbm.at[src_v.at[j1 + 2]], ra_v.at[1], sa1)
                pltpu.make_async_copy(rb_v.at[1], gb_hbm.at[pl.ds(base, 128)], swb1).wait()
                pltpu.async_copy(b_hbm.at[dst_v.at[j1 + 2]], rb_v.at[1], sb1)

            return carry

        lax.fori_loop(0, nch // 2, body, 0)

        @pl.when(nch > 0)
        def _():
            pltpu.make_async_copy(ra_v.at[0], ga_hbm.at[pl.ds(base, 128)], swa0).wait()
            pltpu.make_async_copy(ra_v.at[1], ga_hbm.at[pl.ds(base, 128)], swa1).wait()
            pltpu.make_async_copy(rb_v.at[0], gb_hbm.at[pl.ds(base, 128)], swb0).wait()
            pltpu.make_async_copy(rb_v.at[1], gb_hbm.at[pl.ds(base, 128)], swb1).wait()

    return deg_kernel, agg_kernel, edge_kernel


def _t1_body(degp_ref, x_ref, w_ref, dinv_ref, y_ref):
    i = pl.program_id(0)
    dp = degp_ref[...]
    deg = dp[0, :, :16] + dp[1, :, :16] + 1.0
    rows = lax.broadcasted_iota(jnp.int32, (RB, 16), 0) + i * RB
    dinv = jnp.where(rows < NN, lax.rsqrt(deg), 0.0)
    dinv_ref[...] = dinv
    y_ref[...] = (x_ref[...] @ w_ref[...]) * dinv[:, :1]


def _t2_body(aggp_ref, y_ref, dinv_ref, b_ref, t_ref, stats_ref):
    i = pl.program_id(0)
    ap = aggp_ref[...]
    tt = (ap[0] + ap[1] + y_ref[...]) * dinv_ref[...][:, :1] + b_ref[...][None, :]
    rows = lax.broadcasted_iota(jnp.int32, (RB, H), 0) + i * RB
    ttm = jnp.where(rows < NN, tt, 0.0)
    s0 = jnp.sum(ttm, axis=0)
    s1 = jnp.sum(ttm * ttm, axis=0)
    st = jnp.concatenate([s0[None, :], s1[None, :]], axis=0)
    t_ref[...] = tt

    @pl.when(i == 0)
    def _():
        stats_ref[...] = st

    @pl.when(i > 0)
    def _():
        stats_ref[...] = stats_ref[...] + st


def _bn_relu(t_ref, stats_ref, g_ref, be_ref):
    st = stats_ref[...]
    m = st[0] / NN
    v = st[1] / NN - m * m
    s = lax.rsqrt(v + 1e-5) * g_ref[...]
    c = be_ref[...] - m * s
    return jnp.maximum(t_ref[...] * s[None, :] + c[None, :], 0.0)


def _t3_body(t_ref, stats_ref, g_ref, be_ref, w_ref, dinv_ref, y_ref):
    xb = _bn_relu(t_ref, stats_ref, g_ref, be_ref)
    y_ref[...] = (xb @ w_ref[...]) * dinv_ref[...][:, :1]


def _t7_body(t_ref, stats_ref, g_ref, be_ref, wc0_ref, bc0_ref, a_ref, b_ref):
    xb = _bn_relu(t_ref, stats_ref, g_ref, be_ref)
    w = wc0_ref[...]
    a_ref[...] = xb @ w[:D, :] + bc0_ref[...][None, :]
    b_ref[...] = xb @ w[D:, :]


def _t8_body(ga_ref, gb_ref, wc1_ref, bc1_ref, wc2_ref, bc2_ref, o_ref):
    h0 = jnp.maximum(ga_ref[...] + gb_ref[...], 0.0)
    h1 = jnp.maximum(h0 @ wc1_ref[...] + bc1_ref[...][None, :], 0.0)
    o_ref[...] = h1 @ wc2_ref[...] + bc2_ref[...][None, :]


def _full(shape):
    nd = len(shape)
    return pl.BlockSpec(shape, lambda i: (0,) * nd)


def kernel(x, edge_index, W0, b0, g0, be0, W1, b1, g1, be1, W2, b2, g2, be2,
           Wc0, bc0, Wc1, bc1, Wc2, bc2):
    f32 = jnp.float32
    x_p = jnp.pad(x, ((0, NP - NN), (0, 0)))
    src = jnp.pad(edge_index[0], (0, NCHP * 128 - E), constant_values=NN).reshape(NCHP, 128)
    dst = jnp.pad(edge_index[1], (0, NCHP * 128 - E), constant_values=NN).reshape(NCHP, 128)
    zeros_h = jnp.zeros((RPT, H), f32)
    ones_h = jnp.ones((128, H), f32)

    ngrid = NP // RB

    _deg_kernel, _agg_kernel, _edge_kernel = _sc_kernels()

    degp = _deg_kernel(dst, zeros_h, ones_h)

    dinv, y = pl.pallas_call(
        _t1_body,
        grid=(ngrid,),
        in_specs=[
            pl.BlockSpec((2, RB, H), lambda i: (0, i, 0)),
            pl.BlockSpec((RB, D), lambda i: (i, 0)),
            _full((D, H)),
        ],
        out_specs=[
            pl.BlockSpec((RB, 16), lambda i: (i, 0)),
            pl.BlockSpec((RB, H), lambda i: (i, 0)),
        ],
        out_shape=[
            jax.ShapeDtypeStruct((NP, 16), f32),
            jax.ShapeDtypeStruct((NP, H), f32),
        ],
    )(degp, x_p, W0)

    t2_call = pl.pallas_call(
        _t2_body,
        grid=(ngrid,),
        in_specs=[
            pl.BlockSpec((2, RB, H), lambda i: (0, i, 0)),
            pl.BlockSpec((RB, H), lambda i: (i, 0)),
            pl.BlockSpec((RB, 16), lambda i: (i, 0)),
            _full((H,)),
        ],
        out_specs=[
            pl.BlockSpec((RB, H), lambda i: (i, 0)),
            _full((2, H)),
        ],
        out_shape=[
            jax.ShapeDtypeStruct((NP, H), f32),
            jax.ShapeDtypeStruct((2, H), f32),
        ],
    )

    t3_call = pl.pallas_call(
        _t3_body,
        grid=(ngrid,),
        in_specs=[
            pl.BlockSpec((RB, H), lambda i: (i, 0)),
            _full((2, H)),
            _full((H,)),
            _full((H,)),
            _full((H, H)),
            pl.BlockSpec((RB, 16), lambda i: (i, 0)),
        ],
        out_specs=pl.BlockSpec((RB, H), lambda i: (i, 0)),
        out_shape=jax.ShapeDtypeStruct((NP, H), f32),
    )

    for lidx, (bl, gl, bel, Wn) in enumerate(
            ((b0, g0, be0, W1), (b1, g1, be1, W2))):
        aggp = _agg_kernel(y, src, dst, zeros_h)
        t, stats = t2_call(aggp, y, dinv, bl)
        y = t3_call(t, stats, gl, bel, Wn, dinv)

    aggp = _agg_kernel(y, src, dst, zeros_h)
    t, stats = t2_call(aggp, y, dinv, b2)

    A, B = pl.pallas_call(
        _t7_body,
        grid=(ngrid,),
        in_specs=[
            pl.BlockSpec((RB, H), lambda i: (i, 0)),
            _full((2, H)),
            _full((H,)),
            _full((H,)),
            _full((2 * H, H)),
            _full((H,)),
        ],
        out_specs=[
            pl.BlockSpec((RB, H), lambda i: (i, 0)),
            pl.BlockSpec((RB, H), lambda i: (i, 0)),
        ],
        out_shape=[
            jax.ShapeDtypeStruct((NP, H), f32),
            jax.ShapeDtypeStruct((NP, H), f32),
        ],
    )(t, stats, g2, be2, Wc0, bc0)

    ga, gb = _edge_kernel(A, B, src, dst)

    out = pl.pallas_call(
        _t8_body,
        grid=(EPAD // EB,),
        in_specs=[
            pl.BlockSpec((EB, H), lambda i: (i, 0)),
            pl.BlockSpec((EB, H), lambda i: (i, 0)),
            _full((H, H // 2)),
            _full((H // 2,)),
            _full((H // 2, 2)),
            _full((2,)),
        ],
        out_specs=pl.BlockSpec((EB, 2), lambda i: (i, 0)),
        out_shape=jax.ShapeDtypeStruct((EPAD, 2), f32),
    )(ga, gb, Wc1, bc1, Wc2, bc2)

    return out[:E]
